# baseline (speedup 1.0000x reference)
import functools

import jax
import jax.numpy as jnp
from jax import lax
from jax.experimental import pallas as pl
from jax.experimental.pallas import tpu as pltpu
from jax.experimental.pallas import tpu_sc as plsc

_V = 1000000
_D = 32
_BATCH = 16384
_S = 50
_NW = 32
_BPW = _BATCH // _NW
_G = _S

_mesh = plsc.VectorSubcoreMesh(core_axis_name="c", subcore_axis_name="s")


@functools.partial(
    pl.kernel,
    mesh=_mesh,
    out_type=jax.ShapeDtypeStruct((_S * (_D // 8) * (_BATCH // 512), 4096),
                                  jnp.float32),
    compiler_params=pltpu.CompilerParams(use_tc_tiling_on_sc=False,
                                         needs_layout_passes=False),
    scratch_types=[
        pltpu.VMEM((_BPW * _S,), jnp.int32),
        pltpu.VMEM((2, _BPW), jnp.int32),
        pltpu.VMEM((2, _BPW, _D), jnp.float32),
        pltpu.VMEM((2, 16384), jnp.float32),
        pltpu.SemaphoreType.DMA,
        pltpu.SemaphoreType.DMA((2,)),
        pltpu.SemaphoreType.DMA((2,)),
    ],
)
def _emb_kernel(idx_hbm, table_hbm, out_hbm, xblk, idxl, rows, tiles,
                sem_x, sem_g, sem_o):
    wid = lax.axis_index("s") * 2 + lax.axis_index("c")
    b0 = wid * _BPW

    pltpu.async_copy(
        idx_hbm.at[pl.ds(pl.multiple_of(b0 * _S, _BPW * _S), _BPW * _S)],
        xblk, sem_x).wait()

    def build_idx(g, b):
        lane50 = lax.iota(jnp.int32, 16) * _S
        for cb in range(_BPW // 16):
            v = plsc.load_gather(xblk, [lane50 + (cb * 16 * _S + g)])
            idxl[b, pl.ds(cb * 16, 16)] = v

    def gather_start(g, b):
        pltpu.async_copy(table_hbm.at[idxl.at[b]], rows.at[b], sem_g.at[b])

    def gather_wait(g, b):
        pltpu.make_async_copy(table_hbm.at[idxl.at[b]], rows.at[b],
                              sem_g.at[b]).wait()

    def transpose(b):
        @plsc.parallel_loop(0, _BPW, step=1, unroll=2)
        def _(c):
            lane = lax.iota(jnp.int32, 16)
            jpat = (lane // 8) * 4096 + (lane % 8) * 128
            off = (c // 128) * 1024 + (c % 128)
            v0 = rows[b, c, pl.ds(0, 16)]
            plsc.store_scatter(tiles.at[b], [jpat + off], v0)
            v1 = rows[b, c, pl.ds(16, 16)]
            plsc.store_scatter(tiles.at[b], [jpat + (off + 8192)], v1)

    def store_start(g, b):
        for t1 in range(4):
            pltpu.async_copy(tiles.at[b, pl.ds(t1 * 4096, 4096)],
                             out_hbm.at[(g * 4 + t1) * 32 + wid], sem_o.at[b])

    def store_wait(g, b):
        for t1 in range(4):
            pltpu.make_async_copy(tiles.at[b, pl.ds(t1 * 4096, 4096)],
                                  out_hbm.at[(g * 4 + t1) * 32 + wid],
                                  sem_o.at[b]).wait()

    build_idx(0, 0)
    gather_start(0, 0)

    def body(t, carry):
        for b in range(2):
            g = t * 2 + b
            bn = 1 - b
            gather_wait(g, b)

            @pl.when(g + 1 < _G)
            def _():
                build_idx(g + 1, bn)
                gather_start(g + 1, bn)

            @pl.when(g >= 2)
            def _():
                store_wait(g - 2, b)

            transpose(b)
            store_start(g, b)
        return carry

    lax.fori_loop(0, _G // 2, body, 0)

    store_wait(_G - 2, 0)
    store_wait(_G - 1, 1)


def kernel(x, weight):
    idx = x.reshape(-1)
    out4 = _emb_kernel(idx, weight)
    a5 = out4.reshape(_S, _D // 8, _BATCH // 128, 8, 128)
    out = a5.transpose((2, 4, 0, 1, 3)).reshape(_BATCH, _S, _D)
    return out


# speedup vs baseline: 1.0310x; 1.0310x over previous
import functools

import jax
import jax.numpy as jnp
from jax import lax
from jax.experimental import pallas as pl
from jax.experimental.pallas import tpu as pltpu
from jax.experimental.pallas import tpu_sc as plsc

_V = 1000000
_D = 32
_BATCH = 16384
_S = 50
_NW = 32
_BPW = _BATCH // _NW
_G = _S

_mesh = plsc.VectorSubcoreMesh(core_axis_name="c", subcore_axis_name="s")


@functools.partial(
    pl.kernel,
    mesh=_mesh,
    out_type=jax.ShapeDtypeStruct((_S * (_D // 8) * (_BATCH // 512), 4096),
                                  jnp.float32),
    compiler_params=pltpu.CompilerParams(use_tc_tiling_on_sc=False,
                                         needs_layout_passes=False),
    scratch_types=[
        pltpu.VMEM((_BPW * _S,), jnp.int32),
        pltpu.VMEM((2, _BPW), jnp.int32),
        pltpu.VMEM((2, _BPW, _D), jnp.float32),
        pltpu.VMEM((2, 4, 4096), jnp.float32),
        pltpu.SemaphoreType.DMA,
        pltpu.SemaphoreType.DMA((2,)),
        pltpu.SemaphoreType.DMA((2,)),
    ],
)
def _emb_kernel(idx_hbm, table_hbm, out_hbm, xblk, idxl, rows, tiles,
                sem_x, sem_g, sem_o):
    wid = lax.axis_index("s") * 2 + lax.axis_index("c")
    b0 = wid * _BPW

    pltpu.async_copy(
        idx_hbm.at[pl.ds(pl.multiple_of(b0 * _S, _BPW * _S), _BPW * _S)],
        xblk, sem_x).wait()

    def build_idx(g, b):
        lane50 = lax.iota(jnp.int32, 16) * _S
        for cb in range(_BPW // 16):
            v = plsc.load_gather(xblk, [lane50 + (cb * 16 * _S + g)])
            idxl[b, pl.ds(cb * 16, 16)] = v

    def gather_start(g, b):
        pltpu.async_copy(table_hbm.at[idxl.at[b]], rows.at[b], sem_g.at[b])

    def gather_wait(g, b):
        pltpu.make_async_copy(table_hbm.at[idxl.at[b]], rows.at[b],
                              sem_g.at[b]).wait()

    def transpose(b):
        @plsc.parallel_loop(0, _D, step=1, unroll=4)
        def _(j):
            lane = lax.iota(jnp.int32, 16)
            jv = jnp.full((16,), j, jnp.int32)
            t1 = j // 8
            r128 = (j % 8) * 128
            for t2l in range(4):
                for cb in range(8):
                    c = t2l * 128 + cb * 16
                    v = plsc.load_gather(rows.at[b], [lane + c, jv])
                    tiles[b, t1, pl.ds(r128 + (t2l * 1024 + cb * 16), 16)] = v

    def store_start(g, b):
        for t1 in range(4):
            pltpu.async_copy(tiles.at[b, t1],
                             out_hbm.at[(g * 4 + t1) * 32 + wid], sem_o.at[b])

    def store_wait(g, b):
        for t1 in range(4):
            pltpu.make_async_copy(tiles.at[b, t1],
                                  out_hbm.at[(g * 4 + t1) * 32 + wid],
                                  sem_o.at[b]).wait()

    build_idx(0, 0)
    gather_start(0, 0)

    def body(t, carry):
        for b in range(2):
            g = t * 2 + b
            bn = 1 - b
            gather_wait(g, b)

            @pl.when(g + 1 < _G)
            def _():
                build_idx(g + 1, bn)
                gather_start(g + 1, bn)

            @pl.when(g >= 2)
            def _():
                store_wait(g - 2, b)

            transpose(b)
            store_start(g, b)
        return carry

    lax.fori_loop(0, _G // 2, body, 0)

    store_wait(_G - 2, 0)
    store_wait(_G - 1, 1)


def kernel(x, weight):
    idx = x.reshape(-1)
    out4 = _emb_kernel(idx, weight)
    a5 = out4.reshape(_S, _D // 8, _BATCH // 128, 8, 128)
    out = a5.transpose((2, 4, 0, 1, 3)).reshape(_BATCH, _S, _D)
    return out
